# Initial kernel scaffold; baseline (speedup 1.0000x reference)
import jax
import jax.numpy as jnp
from jax import lax
from jax.experimental import pallas as pl
from jax.experimental.pallas import tpu as pltpu
from jax.experimental.pallas import tpu_sc as plsc

_VOCAB = 100000
_HID = 768
_TOTAL = 1024 * 50
_NW = 32
_BPW = _TOTAL // _NW
_BLK = 64
_NBLK = _BPW // _BLK
_NVEC = _BPW // 16


def _body(x_hbm, orig_hbm, new_hbm, out_hbm,
          idx_v, idx_safe, fix_idx, fix_pos, rows0, rows1, rowbuf,
          gsem0, gsem1, wsem0, wsem1):
    wid = lax.axis_index("s") * 2 + lax.axis_index("c")
    base = wid * _BPW

    pltpu.sync_copy(x_hbm.at[pl.ds(base, _BPW)], idx_v)

    def classify(j, cnt):
        v = idx_v[pl.ds(j * 16, 16)]
        is_new = v >= _VOCAB
        idx_safe[pl.ds(j * 16, 16)] = jnp.where(is_new, 0, v)
        plsc.store_compressed(fix_idx.at[pl.ds(cnt, 16)],
                              v - (_VOCAB - 1), mask=is_new)
        plsc.store_compressed(fix_pos.at[pl.ds(cnt, 16)],
                              j * 16 + lax.iota(jnp.int32, 16), mask=is_new)
        return cnt + jnp.sum(is_new.astype(jnp.int32))

    cnt = lax.fori_loop(0, _NVEC, classify, jnp.int32(0))

    bufs = (rows0, rows1)
    gsems = (gsem0, gsem1)
    wsems = (wsem0, wsem1)

    def g_copy(blk):
        p = blk & 1
        return pltpu.make_async_copy(
            orig_hbm.at[idx_safe.at[pl.ds(blk * _BLK, _BLK)]],
            bufs[p], gsems[p])

    def w_copy(blk):
        p = blk & 1
        return pltpu.make_async_copy(
            bufs[p], out_hbm.at[pl.ds(base + blk * _BLK, _BLK)], wsems[p])

    for blk in range(_NBLK):
        if blk >= 2:
            w_copy(blk - 2).wait()
        g_copy(blk).start()
        if blk >= 1:
            g_copy(blk - 1).wait()
            w_copy(blk - 1).start()
    g_copy(_NBLK - 1).wait()
    w_copy(_NBLK - 1).start()
    w_copy(_NBLK - 2).wait()
    w_copy(_NBLK - 1).wait()

    def fixup(k, carry):
        nid = fix_idx[k]
        pos = fix_pos[k]
        pltpu.sync_copy(new_hbm.at[pl.ds(nid, 1)], rowbuf)
        pltpu.sync_copy(rowbuf, out_hbm.at[pl.ds(base + pos, 1)])
        return carry

    lax.fori_loop(0, cnt, fixup, jnp.int32(0))


_gather = pl.kernel(
    _body,
    out_type=jax.ShapeDtypeStruct((_TOTAL, _HID), jnp.float32),
    mesh=plsc.VectorSubcoreMesh(core_axis_name="c", subcore_axis_name="s"),
    scratch_types=[
        pltpu.VMEM((_BPW,), jnp.int32),
        pltpu.VMEM((_BPW,), jnp.int32),
        pltpu.VMEM((_BPW + 16,), jnp.int32),
        pltpu.VMEM((_BPW + 16,), jnp.int32),
        pltpu.VMEM((_BLK, _HID), jnp.float32),
        pltpu.VMEM((_BLK, _HID), jnp.float32),
        pltpu.VMEM((1, _HID), jnp.float32),
        pltpu.SemaphoreType.DMA,
        pltpu.SemaphoreType.DMA,
        pltpu.SemaphoreType.DMA,
        pltpu.SemaphoreType.DMA,
    ],
)


def kernel(x, orig_table, new_table):
    out = _gather(x.reshape(-1), orig_table, new_table)
    return out.reshape(x.shape[0], x.shape[1], _HID)


# speedup vs baseline: 1.7497x; 1.7497x over previous
import jax
import jax.numpy as jnp
from jax import lax
from jax.experimental import pallas as pl
from jax.experimental.pallas import tpu as pltpu
from jax.experimental.pallas import tpu_sc as plsc

_VOCAB = 100000
_HID = 768
_TOTAL = 1024 * 50
_NW = 32
_BPW = _TOTAL // _NW
_BLK = 64
_NBLK = _BPW // _BLK
_NVEC = _BPW // 16


def _body(x_hbm, orig_hbm, new_hbm, out_hbm,
          idx_v, idx_safe, rows0, rows1, rowbuf,
          gsem0, gsem1, wsem0, wsem1):
    wid = lax.axis_index("s") * 2 + lax.axis_index("c")
    base = wid * _BPW

    pltpu.sync_copy(x_hbm.at[pl.ds(base, _BPW)], idx_v.at[pl.ds(0, _BPW)])

    def clamp(j, carry):
        v = idx_v[pl.ds(j * 16, 16)]
        idx_safe[pl.ds(j * 16, 16)] = jnp.where(v >= _VOCAB, 0, v)
        return carry

    lax.fori_loop(0, _NVEC, clamp, jnp.int32(0))

    def scan_block(blk):
        def fix_lane(k, c2):
            pos = blk * _BLK + k
            s = idx_v[pl.ds(pos, 16)][0]

            @pl.when(s >= _VOCAB)
            def _():
                pltpu.sync_copy(new_hbm.at[pl.ds(s - (_VOCAB - 1), 1)],
                                rowbuf)
                pltpu.sync_copy(rowbuf, out_hbm.at[pl.ds(base + pos, 1)])

            return c2

        lax.fori_loop(0, _BLK, fix_lane, jnp.int32(0))

    bufs = (rows0, rows1)
    gsems = (gsem0, gsem1)
    wsems = (wsem0, wsem1)

    def g_copy(blk):
        p = blk & 1
        return pltpu.make_async_copy(
            orig_hbm.at[idx_safe.at[pl.ds(blk * _BLK, _BLK)]],
            bufs[p], gsems[p])

    def w_copy(blk):
        p = blk & 1
        return pltpu.make_async_copy(
            bufs[p], out_hbm.at[pl.ds(base + blk * _BLK, _BLK)], wsems[p])

    for blk in range(_NBLK):
        if blk >= 2:
            w_copy(blk - 2).wait()
        g_copy(blk).start()
        if blk >= 1:
            g_copy(blk - 1).wait()
            w_copy(blk - 1).start()
        if blk >= 2:
            scan_block(blk - 2)
    g_copy(_NBLK - 1).wait()
    w_copy(_NBLK - 1).start()
    w_copy(_NBLK - 2).wait()
    scan_block(_NBLK - 2)
    w_copy(_NBLK - 1).wait()
    scan_block(_NBLK - 1)


_gather = pl.kernel(
    _body,
    out_type=jax.ShapeDtypeStruct((_TOTAL, _HID), jnp.float32),
    mesh=plsc.VectorSubcoreMesh(core_axis_name="c", subcore_axis_name="s"),
    scratch_types=[
        pltpu.VMEM((_BPW + 16,), jnp.int32),
        pltpu.VMEM((_BPW,), jnp.int32),
        pltpu.VMEM((_BLK, _HID), jnp.float32),
        pltpu.VMEM((_BLK, _HID), jnp.float32),
        pltpu.VMEM((1, _HID), jnp.float32),
        pltpu.SemaphoreType.DMA,
        pltpu.SemaphoreType.DMA,
        pltpu.SemaphoreType.DMA,
        pltpu.SemaphoreType.DMA,
    ],
)


def kernel(x, orig_table, new_table):
    out = _gather(x.reshape(-1), orig_table, new_table)
    return out.reshape(x.shape[0], x.shape[1], _HID)
